# baseline (speedup 1.0000x reference)
import functools

import numpy as np
import jax
import jax.numpy as jnp
from jax import lax
from jax.experimental import pallas as pl
from jax.experimental.pallas import tpu as pltpu
from jax.experimental.pallas import tpu_sc as plsc

B, F = 4096, 1000
TILE = 512
NSTEPS = B // TILE

NC, NS = 2, 16
NW = NC * NS
QROWS = B // NW
ZCH = 8

_ROTS = ((13, 15, 26, 6), (17, 29, 16, 24))


def _np_threefry_pair(k0, k1, x0, x1):
    k0 = np.uint32(k0); k1 = np.uint32(k1)
    ks = (k0, k1, np.uint32(k0 ^ k1 ^ np.uint32(0x1BD11BDA)))
    x0 = np.uint32((int(x0) + int(k0)) & 0xFFFFFFFF)
    x1 = np.uint32((int(x1) + int(k1)) & 0xFFFFFFFF)
    for d in range(5):
        for r in _ROTS[d % 2]:
            x0 = np.uint32((int(x0) + int(x1)) & 0xFFFFFFFF)
            x1 = np.uint32((x1 << np.uint32(r)) | (x1 >> np.uint32(32 - r)))
            x1 = np.uint32(x1 ^ x0)
        x0 = np.uint32((int(x0) + int(ks[(d + 1) % 3])) & 0xFFFFFFFF)
        x1 = np.uint32((int(x1) + int(ks[(d + 2) % 3]) + d + 1) & 0xFFFFFFFF)
    return int(x0), int(x1)


_KINIT = _np_threefry_pair(0, 42, 0, 0)
_KSTEP = _np_threefry_pair(0, 42, 0, 1)


def _tf_fold_bits(keypair, ctr):
    k0, k1 = keypair
    ks = (jnp.uint32(k0), jnp.uint32(k1), jnp.uint32(k0 ^ k1 ^ 0x1BD11BDA))
    x0 = jnp.zeros_like(ctr) + ks[0]
    x1 = ctr + ks[1]
    for d in range(5):
        for r in _ROTS[d % 2]:
            x0 = x0 + x1
            x1 = (x1 << r) | (x1 >> (32 - r))
            x1 = x1 ^ x0
        x0 = x0 + ks[(d + 1) % 3]
        x1 = x1 + ks[(d + 2) % 3] + jnp.uint32(d + 1)
    return x0 ^ x1


def _body(rew_hbm, choice_ref, gr_ref, buf, sems):
    i = pl.program_id(0)
    slot = lax.rem(i, 2)
    nslot = lax.rem(i + 1, 2)

    @pl.when(i == 0)
    def _():
        pltpu.make_async_copy(
            rew_hbm.at[pl.ds(0, TILE)], buf.at[0], sems.at[0]).start()

    @pl.when(i + 1 < NSTEPS)
    def _():
        pltpu.make_async_copy(
            rew_hbm.at[pl.ds((i + 1) * TILE, TILE)], buf.at[nslot],
            sems.at[nslot]).start()

    rows = lax.broadcasted_iota(jnp.int32, (TILE, F), 0)
    fio = lax.broadcasted_iota(jnp.int32, (TILE, F), 1)
    ctr = ((i * TILE + rows) * F + fio).astype(jnp.uint32)

    v1 = (_tf_fold_bits(_KINIT, ctr) >> 9).astype(jnp.int32)
    m1 = jnp.max(v1, axis=1, keepdims=True)
    c1 = jnp.min(jnp.where(v1 == m1, fio, F), axis=1, keepdims=True)

    pltpu.make_async_copy(
        rew_hbm.at[pl.ds(i * TILE, TILE)], buf.at[slot], sems.at[slot]).wait()
    rew = buf[slot]

    gr = jnp.max(jnp.where(fio == c1, rew, -jnp.inf), axis=1, keepdims=True)
    choice_ref[...] = c1
    gr_ref[...] = gr

    neg = gr == -1.0

    @pl.when(jnp.any(neg))
    def _():
        v2 = (_tf_fold_bits(_KSTEP, ctr) >> 9).astype(jnp.int32)
        v2 = jnp.where(fio == c1, -1, v2)
        m2 = jnp.max(v2, axis=1, keepdims=True)
        c2 = jnp.min(jnp.where(v2 == m2, fio, F), axis=1, keepdims=True)
        choice_ref[...] = jnp.where(neg, c2, c1)


_sc_mesh = plsc.VectorSubcoreMesh(core_axis_name="c", subcore_axis_name="s")


@functools.partial(
    pl.kernel,
    mesh=_sc_mesh,
    out_type=jax.ShapeDtypeStruct((B, F), jnp.float32),
    scratch_types=[pltpu.VMEM((ZCH, F), jnp.float32)],
)
def _sc_zeros(z_hbm, q_hbm, zbuf):
    wid = lax.axis_index("s") * NC + lax.axis_index("c")
    base = wid * QROWS
    pltpu.sync_copy(z_hbm, zbuf)
    for k in range(QROWS // ZCH):
        pltpu.sync_copy(zbuf, q_hbm.at[pl.ds(base + k * ZCH, ZCH)])


def kernel(agent_embedding, agent_cell, features, rewards, eval_true=0):
    choice, gr = pl.pallas_call(
        _body,
        grid=(NSTEPS,),
        in_specs=[pl.BlockSpec(memory_space=pl.ANY)],
        out_specs=[
            pl.BlockSpec((TILE, 1), lambda i: (i, 0)),
            pl.BlockSpec((TILE, 1), lambda i: (i, 0)),
        ],
        out_shape=[
            jax.ShapeDtypeStruct((B, 1), jnp.int32),
            jax.ShapeDtypeStruct((B, 1), jnp.float32),
        ],
        scratch_shapes=[
            pltpu.VMEM((2, TILE, F), jnp.float32),
            pltpu.SemaphoreType.DMA((2,)),
        ],
        compiler_params=pltpu.CompilerParams(
            dimension_semantics=("arbitrary",),
        ),
    )(rewards)
    Q = _sc_zeros(jnp.zeros((ZCH, F), dtype=jnp.float32))
    return (Q, choice, gr)


# speedup vs baseline: 1.2861x; 1.2861x over previous
import numpy as np
import jax
import jax.numpy as jnp
from jax import lax
from jax.experimental import pallas as pl
from jax.experimental.pallas import tpu as pltpu

B, F = 4096, 1000
TILE = 512
NSTEPS = B // TILE

_ROTS = ((13, 15, 26, 6), (17, 29, 16, 24))


def _np_threefry_pair(k0, k1, x0, x1):
    k0 = np.uint32(k0); k1 = np.uint32(k1)
    ks = (k0, k1, np.uint32(k0 ^ k1 ^ np.uint32(0x1BD11BDA)))
    x0 = np.uint32((int(x0) + int(k0)) & 0xFFFFFFFF)
    x1 = np.uint32((int(x1) + int(k1)) & 0xFFFFFFFF)
    for d in range(5):
        for r in _ROTS[d % 2]:
            x0 = np.uint32((int(x0) + int(x1)) & 0xFFFFFFFF)
            x1 = np.uint32((x1 << np.uint32(r)) | (x1 >> np.uint32(32 - r)))
            x1 = np.uint32(x1 ^ x0)
        x0 = np.uint32((int(x0) + int(ks[(d + 1) % 3])) & 0xFFFFFFFF)
        x1 = np.uint32((int(x1) + int(ks[(d + 2) % 3]) + d + 1) & 0xFFFFFFFF)
    return int(x0), int(x1)


_KINIT = _np_threefry_pair(0, 42, 0, 0)
_KSTEP = _np_threefry_pair(0, 42, 0, 1)


def _tf_fold_bits(keypair, ctr):
    k0, k1 = keypair
    ks = (jnp.uint32(k0), jnp.uint32(k1), jnp.uint32(k0 ^ k1 ^ 0x1BD11BDA))
    x0 = jnp.zeros_like(ctr) + ks[0]
    x1 = ctr + ks[1]
    for d in range(5):
        for r in _ROTS[d % 2]:
            x0 = x0 + x1
            x1 = (x1 << r) | (x1 >> (32 - r))
            x1 = x1 ^ x0
        x0 = x0 + ks[(d + 1) % 3]
        x1 = x1 + ks[(d + 2) % 3] + jnp.uint32(d + 1)
    return x0 ^ x1


def _body(rew_hbm, choice_ref, gr_ref, buf, sems):
    i = pl.program_id(0)
    slot = i

    @pl.when(i == 0)
    def _():
        for j in range(NSTEPS):
            pltpu.make_async_copy(
                rew_hbm.at[pl.ds(j * TILE, TILE)], buf.at[j], sems.at[j]).start()

    rows = lax.broadcasted_iota(jnp.int32, (TILE, F), 0)
    fio = lax.broadcasted_iota(jnp.int32, (TILE, F), 1)
    ctr = ((i * TILE + rows) * F + fio).astype(jnp.uint32)

    v1 = (_tf_fold_bits(_KINIT, ctr) >> 9).astype(jnp.int32)
    m1 = jnp.max(v1, axis=1, keepdims=True)
    c1 = jnp.min(jnp.where(v1 == m1, fio, F), axis=1, keepdims=True)

    pltpu.make_async_copy(
        rew_hbm.at[pl.ds(i * TILE, TILE)], buf.at[slot], sems.at[slot]).wait()
    rew = buf[slot]

    gr = jnp.max(jnp.where(fio == c1, rew, -jnp.inf), axis=1, keepdims=True)
    choice_ref[...] = c1
    gr_ref[...] = gr

    neg = gr == -1.0

    @pl.when(jnp.any(neg))
    def _():
        v2 = (_tf_fold_bits(_KSTEP, ctr) >> 9).astype(jnp.int32)
        v2 = jnp.where(fio == c1, -1, v2)
        m2 = jnp.max(v2, axis=1, keepdims=True)
        c2 = jnp.min(jnp.where(v2 == m2, fio, F), axis=1, keepdims=True)
        choice_ref[...] = jnp.where(neg, c2, c1)


def kernel(agent_embedding, agent_cell, features, rewards, eval_true=0):
    choice, gr = pl.pallas_call(
        _body,
        grid=(NSTEPS,),
        in_specs=[pl.BlockSpec(memory_space=pl.ANY)],
        out_specs=[
            pl.BlockSpec((TILE, 1), lambda i: (i, 0)),
            pl.BlockSpec((TILE, 1), lambda i: (i, 0)),
        ],
        out_shape=[
            jax.ShapeDtypeStruct((B, 1), jnp.int32),
            jax.ShapeDtypeStruct((B, 1), jnp.float32),
        ],
        scratch_shapes=[
            pltpu.VMEM((NSTEPS, TILE, F), jnp.float32),
            pltpu.SemaphoreType.DMA((NSTEPS,)),
        ],
        compiler_params=pltpu.CompilerParams(
            dimension_semantics=("arbitrary",),
        ),
    )(rewards)
    Q = jnp.zeros((B, F), dtype=jnp.float32)
    return (Q, choice, gr)
